# baseline (speedup 1.0000x reference)
import functools

import jax
import jax.numpy as jnp
from jax import lax
from jax.experimental import pallas as pl
from jax.experimental.pallas import tpu as pltpu
from jax.experimental.pallas import tpu_sc as plsc

_N_TIMES = 1000
_N_LOCS = 1000
_N_APPS = 8000
_N_NODES = 10000
_DIM = 128
_SEQ = 20

_NP = 10240
_TRASH = 10100
_CH = 128
_NSUB = 16
_NW = 32
_ROWS_PW = _NP // _NSUB


def _sc_aggregate(table_p, table_n, src, dst, act_p, act_n, with_deg):
    ep = src.shape[0]
    epw = ep // _NSUB
    nch = epw // _CH
    assert nch % 2 == 0
    mesh = plsc.VectorSubcoreMesh(core_axis_name="c", subcore_axis_name="s")

    out_type = [jax.ShapeDtypeStruct((_NW, _ROWS_PW, _DIM), jnp.float32)]
    scratch = [
        pltpu.VMEM((_CH,), jnp.int32),
        pltpu.VMEM((_CH,), jnp.int32),
        pltpu.VMEM((_CH,), jnp.int32),
        pltpu.VMEM((_CH,), jnp.int32),
        pltpu.VMEM((_CH, _DIM), jnp.float32),
        pltpu.VMEM((_CH, _DIM), jnp.float32),
        pltpu.VMEM_SHARED((_NP, _DIM), jnp.float32),
        pltpu.SemaphoreType.DMA,
        pltpu.SemaphoreType.DMA,
        pltpu.SemaphoreType.DMA,
        pltpu.SemaphoreType.DMA,
    ]
    if with_deg:
        out_type.append(jax.ShapeDtypeStruct((_NW, _ROWS_PW), jnp.int32))
        scratch += [pltpu.VMEM((_CH,), jnp.int32),
                    pltpu.VMEM((_CH,), jnp.int32),
                    pltpu.SemaphoreType.DMA,
                    pltpu.SemaphoreType.DMA,
                    pltpu.VMEM_SHARED((_NP,), jnp.int32)]

    @functools.partial(pl.kernel, out_type=out_type, mesh=mesh,
                       scratch_types=scratch)
    def k(tp_h, tn_h, src_h, dst_h, actp_h, actn_h, zeros_h, zeros1_h,
          *rest):
        if with_deg:
            (out_h, deg_out_h, sva, svb, dva, dvb, rva, rvb, acc_sh,
             s_sta, s_stb, s_ra, s_rb, ava, avb, s_aa, s_ab,
             deg_sh) = rest
        else:
            (out_h, sva, svb, dva, dvb, rva, rvb, acc_sh,
             s_sta, s_stb, s_ra, s_rb) = rest
            ava = avb = s_aa = s_ab = deg_sh = None
        c = lax.axis_index("c")
        s = lax.axis_index("s")
        wid = c * _NSUB + s
        pltpu.sync_copy(zeros_h, acc_sh.at[pl.ds(s * _ROWS_PW, _ROWS_PW)])
        if with_deg:
            pltpu.sync_copy(zeros1_h,
                            deg_sh.at[pl.ds(s * _ROWS_PW, _ROWS_PW)])
        plsc.subcore_barrier()
        base = s * epw

        def stage(off, sv, dv, sem):
            pltpu.async_copy(src_h.at[pl.ds(off, _CH)], sv, sem)
            pltpu.async_copy(dst_h.at[pl.ds(off, _CH)], dv, sem)

        def stage_wait(off, sv, dv, sem):
            pltpu.make_async_copy(src_h.at[pl.ds(off, _CH)], sv, sem).wait()
            pltpu.make_async_copy(dst_h.at[pl.ds(off, _CH)], dv, sem).wait()

        def issue_gathers(sv, av, rv, s_a, s_r):
            @pl.when(c == 0)
            def _():
                pltpu.async_copy(tp_h.at[sv], rv, s_r)
                if with_deg:
                    pltpu.async_copy(actp_h.at[sv], av, s_a)

            @pl.when(c == 1)
            def _():
                pltpu.async_copy(tn_h.at[sv], rv, s_r)
                if with_deg:
                    pltpu.async_copy(actn_h.at[sv], av, s_a)

        def finish(sv, dv, av, rv, s_a, s_r):
            pltpu.make_async_copy(tp_h.at[sv], rv, s_r).wait()
            pltpu.sync_copy(rv, acc_sh.at[dv], add=True)
            if with_deg:
                pltpu.make_async_copy(actp_h.at[sv], av, s_a).wait()
                pltpu.sync_copy(av, deg_sh.at[dv], add=True)

        def body(t, carry):
            off_a = base + (2 * t) * _CH
            off_b = off_a + _CH
            stage(off_a, sva, dva, s_sta)
            stage(off_b, svb, dvb, s_stb)
            stage_wait(off_a, sva, dva, s_sta)
            issue_gathers(sva, ava, rva, s_aa, s_ra)
            stage_wait(off_b, svb, dvb, s_stb)
            issue_gathers(svb, avb, rvb, s_ab, s_rb)
            finish(sva, dva, ava, rva, s_aa, s_ra)
            finish(svb, dvb, avb, rvb, s_ab, s_rb)
            return carry

        lax.fori_loop(0, nch // 2, body, 0)
        plsc.subcore_barrier()
        pltpu.sync_copy(acc_sh.at[pl.ds(s * _ROWS_PW, _ROWS_PW)],
                        out_h.at[wid])
        if with_deg:
            pltpu.sync_copy(deg_sh.at[pl.ds(s * _ROWS_PW, _ROWS_PW)],
                            deg_out_h.at[wid])

    zeros = jnp.zeros((_ROWS_PW, _DIM), jnp.float32)
    zeros1 = jnp.zeros((_ROWS_PW,), jnp.int32)
    if with_deg:
        out, deg = k(table_p, table_n, src, dst, act_p, act_n, zeros,
                     zeros1)
        return (out.reshape(2, _NP, _DIM), deg.reshape(2, _NP))
    (out,) = k(table_p, table_n, src, dst, act_p, act_n, zeros, zeros1)
    return out.reshape(2, _NP, _DIM)


def _sc_gather(table, idx):
    n = idx.shape[0]
    per_w = n // _NW
    nch = per_w // _CH
    assert nch % 2 == 0
    mesh = plsc.VectorSubcoreMesh(core_axis_name="c", subcore_axis_name="s")

    @functools.partial(
        pl.kernel,
        out_type=jax.ShapeDtypeStruct((n, _DIM), jnp.float32),
        mesh=mesh,
        scratch_types=[
            pltpu.VMEM((_CH,), jnp.int32),
            pltpu.VMEM((_CH,), jnp.int32),
            pltpu.VMEM((_CH, _DIM), jnp.float32),
            pltpu.VMEM((_CH, _DIM), jnp.float32),
            pltpu.SemaphoreType.DMA,
            pltpu.SemaphoreType.DMA,
            pltpu.SemaphoreType.DMA,
            pltpu.SemaphoreType.DMA,
        ],
    )
    def k(table_h, idx_h, out_h, iva, ivb, rva, rvb, s_ia, s_ib, s_ra,
          s_rb):
        c = lax.axis_index("c")
        s = lax.axis_index("s")
        base = (c * _NSUB + s) * per_w

        def body(t, carry):
            off_a = base + (2 * t) * _CH
            off_b = off_a + _CH
            pltpu.async_copy(idx_h.at[pl.ds(off_a, _CH)], iva, s_ia)
            pltpu.async_copy(idx_h.at[pl.ds(off_b, _CH)], ivb, s_ib)
            pltpu.make_async_copy(idx_h.at[pl.ds(off_a, _CH)], iva,
                                  s_ia).wait()
            pltpu.async_copy(table_h.at[iva], rva, s_ra)
            pltpu.make_async_copy(idx_h.at[pl.ds(off_b, _CH)], ivb,
                                  s_ib).wait()
            pltpu.async_copy(table_h.at[ivb], rvb, s_rb)
            pltpu.make_async_copy(table_h.at[iva], rva, s_ra).wait()
            pltpu.sync_copy(rva, out_h.at[pl.ds(off_a, _CH)])
            pltpu.make_async_copy(table_h.at[ivb], rvb, s_rb).wait()
            pltpu.sync_copy(rvb, out_h.at[pl.ds(off_b, _CH)])
            return carry

        lax.fori_loop(0, nch // 2, body, 0)

    return k(table, idx)


_RB = 1024


def _mm_body(x_ref, sp_ref, sn_ref, dp_ref, dn_ref, ap_ref, an_ref,
             ws_ref, wn_ref, b_ref, hap_ref, han_ref, hb_ref):
    x = x_ref[...]
    ws = ws_ref[...]
    wn = wn_ref[...]
    b = b_ref[...]
    s = lax.dot_general(x, ws, (((1,), (1,)), ((), ())),
                        preferred_element_type=jnp.float32)
    np_ = sp_ref[...] / jnp.maximum(dp_ref[...][:, 0:1], 1.0)
    nn_ = sn_ref[...] / jnp.maximum(dn_ref[...][:, 0:1], 1.0)
    mp = lax.dot_general(np_, wn, (((1,), (1,)), ((), ())),
                         preferred_element_type=jnp.float32)
    mn = lax.dot_general(nn_, wn, (((1,), (1,)), ((), ())),
                         preferred_element_type=jnp.float32)
    hap_ref[...] = jnp.maximum(s + mp + b, 0.0) * ap_ref[...][:, 0:1]
    han_ref[...] = jnp.maximum(s + mn + b, 0.0) * an_ref[...][:, 0:1]
    hb_ref[...] = jnp.maximum(s + b, 0.0)


def _tc_sage(x, sum_p, sum_n, deg_p, deg_n, act_p16, act_n16, Ws, Wn, b):
    grid = (_NP // _RB,)
    row = pl.BlockSpec((_RB, _DIM), lambda i: (i, 0))
    dcol = pl.BlockSpec((_RB, 16), lambda i: (i, 0))
    full = pl.BlockSpec((_DIM, _DIM), lambda i: (0, 0))
    bspec = pl.BlockSpec((1, _DIM), lambda i: (0, 0))
    out_sh = jax.ShapeDtypeStruct((_NP, _DIM), jnp.float32)
    return pl.pallas_call(
        _mm_body,
        grid=grid,
        in_specs=[row, row, row, dcol, dcol, dcol, dcol, full, full, bspec],
        out_specs=[row, row, row],
        out_shape=[out_sh, out_sh, out_sh],
    )(x, sum_p, sum_n, deg_p, deg_n, act_p16, act_n16, Ws, Wn,
      b.reshape(1, _DIM))


def _mm2_body(xp_ref, xn_ref, xb_ref, sp_ref, sn_ref, dp_ref, dn_ref,
              ws_ref, wn_ref, b_ref, hap_ref, han_ref, hb_ref):
    ws = ws_ref[...]
    wn = wn_ref[...]
    b = b_ref[...]
    dot = lambda a, w: lax.dot_general(a, w, (((1,), (1,)), ((), ())),
                                       preferred_element_type=jnp.float32)
    np_ = sp_ref[...] / jnp.maximum(dp_ref[...][:, 0:1], 1.0)
    nn_ = sn_ref[...] / jnp.maximum(dn_ref[...][:, 0:1], 1.0)
    hap_ref[...] = jnp.maximum(dot(xp_ref[...], ws) + dot(np_, wn) + b, 0.0)
    han_ref[...] = jnp.maximum(dot(xn_ref[...], ws) + dot(nn_, wn) + b, 0.0)
    hb_ref[...] = jnp.maximum(dot(xb_ref[...], ws) + b, 0.0)


def _tc_sage2(xp, xn, xb, sum_p, sum_n, deg_p, deg_n, Ws, Wn, b):
    grid = (_NP // _RB,)
    row = pl.BlockSpec((_RB, _DIM), lambda i: (i, 0))
    dcol = pl.BlockSpec((_RB, 16), lambda i: (i, 0))
    full = pl.BlockSpec((_DIM, _DIM), lambda i: (0, 0))
    bspec = pl.BlockSpec((1, _DIM), lambda i: (0, 0))
    out_sh = jax.ShapeDtypeStruct((_NP, _DIM), jnp.float32)
    return pl.pallas_call(
        _mm2_body,
        grid=grid,
        in_specs=[row, row, row, row, row, dcol, dcol, full, full, bspec],
        out_specs=[row, row, row],
        out_shape=[out_sh, out_sh, out_sh],
    )(xp, xn, xb, sum_p, sum_n, deg_p, deg_n, Ws, Wn, b.reshape(1, _DIM))


_SB = 128
_GRP = 24


def _sim_body(g_ref, out_ref):
    h = g_ref[...]
    t = h[:, 0, :]
    l = h[:, 1, :]
    a = h[:, 2:_SEQ + 2, :]
    nt2 = jnp.sum(t * t, axis=-1, keepdims=True)
    nl2 = jnp.sum(l * l, axis=-1, keepdims=True)
    stl = jnp.sum(t * l, axis=-1, keepdims=True)
    na2 = jnp.sum(a * a, axis=-1)
    p = jnp.sum(t[:, None, :] * a, axis=-1)
    q = jnp.sum(l[:, None, :] * a, axis=-1)
    nt = jnp.sqrt(nt2)
    nl = jnp.sqrt(nl2)
    ntl = jnp.sqrt(nt2 + 2.0 * stl + nl2)
    nu = jnp.sqrt(nl2 + 2.0 * q + na2)
    nv = jnp.sqrt(nt2 + 2.0 * p + na2)
    na = jnp.sqrt(na2)
    sim_t = jnp.sum((stl + p) / (nt * nu), axis=-1)
    sim_l = jnp.sum((stl + q) / (nl * nv), axis=-1)
    sim_a = jnp.sum((p + q) / (na * ntl), axis=-1)
    out_ref[...] = sim_t + sim_l + sim_a


def _tc_sim(g, nrows):
    grid = (nrows // _SB,)
    return pl.pallas_call(
        _sim_body,
        grid=grid,
        in_specs=[pl.BlockSpec((_SB, _GRP, _DIM), lambda i: (i, 0, 0))],
        out_specs=pl.BlockSpec((_SB,), lambda i: (i,)),
        out_shape=jax.ShapeDtypeStruct((nrows,), jnp.float32),
    )(g)


def kernel(users, times, locs, app_seq, edge_index, tla_emb,
           Ws1, Wn1, b1, Ws2, Wn2, b2):
    batch = users.shape[0]
    m = batch * (_SEQ + 2)
    nodes_idx = jnp.concatenate(
        [_N_APPS + _N_LOCS + times, _N_APPS + locs, app_seq],
        axis=1).reshape(-1)
    nk = jax.random.key(42)
    nks = jax.random.split(nk, 4)
    neg_t = jax.random.randint(nks[1], (batch, 1), 0, _N_TIMES)
    neg_l = jax.random.randint(nks[2], (batch, 1), 0, _N_LOCS)
    neg_a = jax.random.randint(nks[3], (batch, _SEQ), 0, _N_APPS)
    neg_nodes_idx = jnp.concatenate(
        [_N_APPS + _N_LOCS + neg_t, _N_APPS + neg_l, neg_a],
        axis=1).reshape(-1)

    ar = jnp.arange(m, dtype=jnp.int32)
    lp_pos = jnp.full((_NP,), -1, jnp.int32).at[nodes_idx].max(ar)
    lp_neg = jnp.full((_NP,), -1, jnp.int32).at[neg_nodes_idx].max(ar)
    act_p = (lp_pos >= 0).astype(jnp.int32)
    act_n = (lp_neg >= 0).astype(jnp.int32)
    islast_p = jnp.zeros((m,), jnp.int32).at[
        jnp.where(lp_pos >= 0, lp_pos, m)].set(1, mode="drop")
    islast_n = jnp.zeros((m,), jnp.int32).at[
        jnp.where(lp_neg >= 0, lp_neg, m)].set(1, mode="drop")

    src, dst = edge_index[0], edge_index[1]
    e = src.shape[0]
    ep = -(-e // (_NSUB * _CH * 2)) * (_NSUB * _CH * 2)
    pad = ep - e
    src_p = jnp.concatenate([src.astype(jnp.int32),
                             jnp.zeros((pad,), jnp.int32)])
    dst_p = jnp.concatenate([dst.astype(jnp.int32),
                             jnp.full((pad,), _TRASH, jnp.int32)])

    x_pad = jnp.zeros((_NP, _DIM), jnp.float32).at[:_N_NODES].set(tla_emb)
    x_mp = x_pad * act_p.astype(jnp.float32)[:, None]
    x_mn = x_pad * act_n.astype(jnp.float32)[:, None]

    acc_a, deg = _sc_aggregate(x_mp, x_mn, src_p, dst_p,
                               act_p, act_n, True)
    sum1_p, sum1_n = acc_a[0], acc_a[1]
    degf = deg.astype(jnp.float32)
    dg_p = jnp.broadcast_to(degf[0][:, None], (_NP, 16))
    dg_n = jnp.broadcast_to(degf[1][:, None], (_NP, 16))
    ap16 = jnp.broadcast_to(act_p.astype(jnp.float32)[:, None], (_NP, 16))
    an16 = jnp.broadcast_to(act_n.astype(jnp.float32)[:, None], (_NP, 16))

    h1a_p, h1a_n, h1b = _tc_sage(x_pad, sum1_p, sum1_n, dg_p, dg_n,
                                 ap16, an16, Ws1, Wn1, b1)

    acc_b = _sc_aggregate(h1a_p, h1a_n, src_p, dst_p, act_p, act_n, False)
    sum2_p, sum2_n = acc_b[0], acc_b[1]

    h2a_p, h2a_n, h2b = _tc_sage2(h1a_p, h1a_n, h1b, sum2_p, sum2_n,
                                  dg_p, dg_n, Ws2, Wn2, b2)

    tf = jnp.concatenate([h2a_p, h2a_n, h2b], axis=0)
    idx_pos = jnp.where(islast_p == 1, nodes_idx, nodes_idx + 2 * _NP)
    idx_neg = jnp.where(islast_n == 1, neg_nodes_idx + _NP,
                        neg_nodes_idx + 2 * _NP)
    idx_all = jnp.concatenate([idx_pos, idx_neg]).astype(jnp.int32)
    idx24 = jnp.pad(idx_all.reshape(2 * batch, _SEQ + 2),
                    ((0, 0), (0, _GRP - _SEQ - 2))).reshape(-1)
    g = _sc_gather(tf, idx24)
    loss = _tc_sim(g.reshape(2 * batch, _GRP, _DIM), 2 * batch)
    return loss[:batch], loss[batch:]


# speedup vs baseline: 1.4625x; 1.4625x over previous
import functools

import jax
import jax.numpy as jnp
from jax import lax
from jax.experimental import pallas as pl
from jax.experimental.pallas import tpu as pltpu
from jax.experimental.pallas import tpu_sc as plsc

_N_TIMES = 1000
_N_LOCS = 1000
_N_APPS = 8000
_N_NODES = 10000
_DIM = 128
_SEQ = 20

_NP = 10240
_TRASH = 10100
_CH = 128
_NSUB = 16
_NW = 32
_ROWS_PW = _NP // _NSUB


def _sc_aggregate(table_p, table_n, src, dst, act_p, act_n, with_deg):
    ep = src.shape[0]
    epw = ep // _NSUB
    nch = epw // _CH
    assert nch % 2 == 0
    mesh = plsc.VectorSubcoreMesh(core_axis_name="c", subcore_axis_name="s")

    out_type = [jax.ShapeDtypeStruct((_NW, _ROWS_PW, _DIM), jnp.float32)]
    scratch = [
        pltpu.VMEM((_CH,), jnp.int32),
        pltpu.VMEM((_CH,), jnp.int32),
        pltpu.VMEM((_CH,), jnp.int32),
        pltpu.VMEM((_CH,), jnp.int32),
        pltpu.VMEM((_CH, _DIM), jnp.float32),
        pltpu.VMEM((_CH, _DIM), jnp.float32),
        pltpu.VMEM_SHARED((_NP, _DIM), jnp.float32),
        pltpu.SemaphoreType.DMA,
        pltpu.SemaphoreType.DMA,
        pltpu.SemaphoreType.DMA,
        pltpu.SemaphoreType.DMA,
    ]
    if with_deg:
        out_type.append(jax.ShapeDtypeStruct((_NW, _ROWS_PW), jnp.int32))
        scratch += [pltpu.VMEM((_CH,), jnp.int32),
                    pltpu.VMEM((_CH,), jnp.int32),
                    pltpu.SemaphoreType.DMA,
                    pltpu.SemaphoreType.DMA,
                    pltpu.VMEM_SHARED((_NP,), jnp.int32)]

    @functools.partial(pl.kernel, out_type=out_type, mesh=mesh,
                       scratch_types=scratch)
    def k(tp_h, tn_h, src_h, dst_h, actp_h, actn_h, zeros_h, zeros1_h,
          *rest):
        if with_deg:
            (out_h, deg_out_h, sva, svb, dva, dvb, rva, rvb, acc_sh,
             s_sta, s_stb, s_ra, s_rb, ava, avb, s_aa, s_ab,
             deg_sh) = rest
        else:
            (out_h, sva, svb, dva, dvb, rva, rvb, acc_sh,
             s_sta, s_stb, s_ra, s_rb) = rest
            ava = avb = s_aa = s_ab = deg_sh = None
        c = lax.axis_index("c")
        s = lax.axis_index("s")
        wid = c * _NSUB + s
        pltpu.sync_copy(zeros_h, acc_sh.at[pl.ds(s * _ROWS_PW, _ROWS_PW)])
        if with_deg:
            pltpu.sync_copy(zeros1_h,
                            deg_sh.at[pl.ds(s * _ROWS_PW, _ROWS_PW)])
        plsc.subcore_barrier()
        base = s * epw

        def stage(off, sv, dv, sem):
            pltpu.async_copy(src_h.at[pl.ds(off, _CH)], sv, sem)
            pltpu.async_copy(dst_h.at[pl.ds(off, _CH)], dv, sem)

        def stage_wait(off, sv, dv, sem):
            pltpu.make_async_copy(src_h.at[pl.ds(off, _CH)], sv, sem).wait()
            pltpu.make_async_copy(dst_h.at[pl.ds(off, _CH)], dv, sem).wait()

        def issue_gathers(sv, av, rv, s_a, s_r):
            @pl.when(c == 0)
            def _():
                pltpu.async_copy(tp_h.at[sv], rv, s_r)
                if with_deg:
                    pltpu.async_copy(actp_h.at[sv], av, s_a)

            @pl.when(c == 1)
            def _():
                pltpu.async_copy(tn_h.at[sv], rv, s_r)
                if with_deg:
                    pltpu.async_copy(actn_h.at[sv], av, s_a)

        def finish(sv, dv, av, rv, s_a, s_r):
            pltpu.make_async_copy(tp_h.at[sv], rv, s_r).wait()
            pltpu.sync_copy(rv, acc_sh.at[dv], add=True)
            if with_deg:
                pltpu.make_async_copy(actp_h.at[sv], av, s_a).wait()
                pltpu.sync_copy(av, deg_sh.at[dv], add=True)

        def body(t, carry):
            off_a = base + (2 * t) * _CH
            off_b = off_a + _CH
            stage(off_a, sva, dva, s_sta)
            stage(off_b, svb, dvb, s_stb)
            stage_wait(off_a, sva, dva, s_sta)
            issue_gathers(sva, ava, rva, s_aa, s_ra)
            stage_wait(off_b, svb, dvb, s_stb)
            issue_gathers(svb, avb, rvb, s_ab, s_rb)
            finish(sva, dva, ava, rva, s_aa, s_ra)
            finish(svb, dvb, avb, rvb, s_ab, s_rb)
            return carry

        lax.fori_loop(0, nch // 2, body, 0)
        plsc.subcore_barrier()
        pltpu.sync_copy(acc_sh.at[pl.ds(s * _ROWS_PW, _ROWS_PW)],
                        out_h.at[wid])
        if with_deg:
            pltpu.sync_copy(deg_sh.at[pl.ds(s * _ROWS_PW, _ROWS_PW)],
                            deg_out_h.at[wid])

    zeros = jnp.zeros((_ROWS_PW, _DIM), jnp.float32)
    zeros1 = jnp.zeros((_ROWS_PW,), jnp.int32)
    if with_deg:
        out, deg = k(table_p, table_n, src, dst, act_p, act_n, zeros,
                     zeros1)
        return (out.reshape(2, _NP, _DIM), deg.reshape(2, _NP))
    (out,) = k(table_p, table_n, src, dst, act_p, act_n, zeros, zeros1)
    return out.reshape(2, _NP, _DIM)


def _sc_gather(table, idx):
    n = idx.shape[0]
    per_w = n // _NW
    nch = per_w // _CH
    assert nch % 2 == 0
    mesh = plsc.VectorSubcoreMesh(core_axis_name="c", subcore_axis_name="s")

    @functools.partial(
        pl.kernel,
        out_type=jax.ShapeDtypeStruct((n, _DIM), jnp.float32),
        mesh=mesh,
        scratch_types=[
            pltpu.VMEM((_CH,), jnp.int32),
            pltpu.VMEM((_CH,), jnp.int32),
            pltpu.VMEM((_CH, _DIM), jnp.float32),
            pltpu.VMEM((_CH, _DIM), jnp.float32),
            pltpu.SemaphoreType.DMA,
            pltpu.SemaphoreType.DMA,
            pltpu.SemaphoreType.DMA,
            pltpu.SemaphoreType.DMA,
        ],
    )
    def k(table_h, idx_h, out_h, iva, ivb, rva, rvb, s_ia, s_ib, s_ra,
          s_rb):
        c = lax.axis_index("c")
        s = lax.axis_index("s")
        base = (c * _NSUB + s) * per_w

        def body(t, carry):
            off_a = base + (2 * t) * _CH
            off_b = off_a + _CH
            pltpu.async_copy(idx_h.at[pl.ds(off_a, _CH)], iva, s_ia)
            pltpu.async_copy(idx_h.at[pl.ds(off_b, _CH)], ivb, s_ib)
            pltpu.make_async_copy(idx_h.at[pl.ds(off_a, _CH)], iva,
                                  s_ia).wait()
            pltpu.async_copy(table_h.at[iva], rva, s_ra)
            pltpu.make_async_copy(idx_h.at[pl.ds(off_b, _CH)], ivb,
                                  s_ib).wait()
            pltpu.async_copy(table_h.at[ivb], rvb, s_rb)
            pltpu.make_async_copy(table_h.at[iva], rva, s_ra).wait()
            pltpu.sync_copy(rva, out_h.at[pl.ds(off_a, _CH)])
            pltpu.make_async_copy(table_h.at[ivb], rvb, s_rb).wait()
            pltpu.sync_copy(rvb, out_h.at[pl.ds(off_b, _CH)])
            return carry

        lax.fori_loop(0, nch // 2, body, 0)

    return k(table, idx)


_RB = 1024


def _mm_body(x_ref, sp_ref, sn_ref, dp_ref, dn_ref, ap_ref, an_ref,
             ws_ref, wn_ref, b_ref, hap_ref, han_ref, hb_ref):
    x = x_ref[...]
    ws = ws_ref[...]
    wn = wn_ref[...]
    b = b_ref[...]
    s = lax.dot_general(x, ws, (((1,), (1,)), ((), ())),
                        preferred_element_type=jnp.float32)
    np_ = sp_ref[...] / jnp.maximum(dp_ref[...][:, 0:1], 1.0)
    nn_ = sn_ref[...] / jnp.maximum(dn_ref[...][:, 0:1], 1.0)
    mp = lax.dot_general(np_, wn, (((1,), (1,)), ((), ())),
                         preferred_element_type=jnp.float32)
    mn = lax.dot_general(nn_, wn, (((1,), (1,)), ((), ())),
                         preferred_element_type=jnp.float32)
    hap_ref[...] = jnp.maximum(s + mp + b, 0.0) * ap_ref[...][:, 0:1]
    han_ref[...] = jnp.maximum(s + mn + b, 0.0) * an_ref[...][:, 0:1]
    hb_ref[...] = jnp.maximum(s + b, 0.0)


def _tc_sage(x, sum_p, sum_n, deg_p, deg_n, act_p16, act_n16, Ws, Wn, b):
    grid = (_NP // _RB,)
    row = pl.BlockSpec((_RB, _DIM), lambda i: (i, 0))
    dcol = pl.BlockSpec((_RB, 16), lambda i: (i, 0))
    full = pl.BlockSpec((_DIM, _DIM), lambda i: (0, 0))
    bspec = pl.BlockSpec((1, _DIM), lambda i: (0, 0))
    out_sh = jax.ShapeDtypeStruct((_NP, _DIM), jnp.float32)
    return pl.pallas_call(
        _mm_body,
        grid=grid,
        in_specs=[row, row, row, dcol, dcol, dcol, dcol, full, full, bspec],
        out_specs=[row, row, row],
        out_shape=[out_sh, out_sh, out_sh],
    )(x, sum_p, sum_n, deg_p, deg_n, act_p16, act_n16, Ws, Wn,
      b.reshape(1, _DIM))


def _mm2_body(xp_ref, xn_ref, xb_ref, sp_ref, sn_ref, dp_ref, dn_ref,
              ws_ref, wn_ref, b_ref, hap_ref, han_ref, hb_ref):
    ws = ws_ref[...]
    wn = wn_ref[...]
    b = b_ref[...]
    dot = lambda a, w: lax.dot_general(a, w, (((1,), (1,)), ((), ())),
                                       preferred_element_type=jnp.float32)
    np_ = sp_ref[...] / jnp.maximum(dp_ref[...][:, 0:1], 1.0)
    nn_ = sn_ref[...] / jnp.maximum(dn_ref[...][:, 0:1], 1.0)
    hap_ref[...] = jnp.maximum(dot(xp_ref[...], ws) + dot(np_, wn) + b, 0.0)
    han_ref[...] = jnp.maximum(dot(xn_ref[...], ws) + dot(nn_, wn) + b, 0.0)
    hb_ref[...] = jnp.maximum(dot(xb_ref[...], ws) + b, 0.0)


def _tc_sage2(xp, xn, xb, sum_p, sum_n, deg_p, deg_n, Ws, Wn, b):
    grid = (_NP // _RB,)
    row = pl.BlockSpec((_RB, _DIM), lambda i: (i, 0))
    dcol = pl.BlockSpec((_RB, 16), lambda i: (i, 0))
    full = pl.BlockSpec((_DIM, _DIM), lambda i: (0, 0))
    bspec = pl.BlockSpec((1, _DIM), lambda i: (0, 0))
    out_sh = jax.ShapeDtypeStruct((_NP, _DIM), jnp.float32)
    return pl.pallas_call(
        _mm2_body,
        grid=grid,
        in_specs=[row, row, row, row, row, dcol, dcol, full, full, bspec],
        out_specs=[row, row, row],
        out_shape=[out_sh, out_sh, out_sh],
    )(xp, xn, xb, sum_p, sum_n, deg_p, deg_n, Ws, Wn, b.reshape(1, _DIM))


_SB = 128
_GRP = 24


def _sim_body(g_ref, out_ref):
    h = g_ref[...]
    t = h[:, 0, :]
    l = h[:, 1, :]
    a = h[:, 2:_SEQ + 2, :]
    nt2 = jnp.sum(t * t, axis=-1, keepdims=True)
    nl2 = jnp.sum(l * l, axis=-1, keepdims=True)
    stl = jnp.sum(t * l, axis=-1, keepdims=True)
    na2 = jnp.sum(a * a, axis=-1)
    p = jnp.sum(t[:, None, :] * a, axis=-1)
    q = jnp.sum(l[:, None, :] * a, axis=-1)
    nt = jnp.sqrt(nt2)
    nl = jnp.sqrt(nl2)
    ntl = jnp.sqrt(nt2 + 2.0 * stl + nl2)
    nu = jnp.sqrt(nl2 + 2.0 * q + na2)
    nv = jnp.sqrt(nt2 + 2.0 * p + na2)
    na = jnp.sqrt(na2)
    sim_t = jnp.sum((stl + p) / (nt * nu), axis=-1)
    sim_l = jnp.sum((stl + q) / (nl * nv), axis=-1)
    sim_a = jnp.sum((p + q) / (na * ntl), axis=-1)
    out_ref[...] = sim_t + sim_l + sim_a


def _tc_sim(g, nrows):
    grid = (nrows // _SB,)
    return pl.pallas_call(
        _sim_body,
        grid=grid,
        in_specs=[pl.BlockSpec((_SB, _GRP, _DIM), lambda i: (i, 0, 0))],
        out_specs=pl.BlockSpec((_SB,), lambda i: (i,)),
        out_shape=jax.ShapeDtypeStruct((nrows,), jnp.float32),
    )(g)


def kernel(users, times, locs, app_seq, edge_index, tla_emb,
           Ws1, Wn1, b1, Ws2, Wn2, b2):
    batch = users.shape[0]
    m = batch * (_SEQ + 2)
    nodes_idx = jnp.concatenate(
        [_N_APPS + _N_LOCS + times, _N_APPS + locs, app_seq],
        axis=1).reshape(-1)
    nk = jax.random.key(42)
    nks = jax.random.split(nk, 4)
    neg_t = jax.random.randint(nks[1], (batch, 1), 0, _N_TIMES)
    neg_l = jax.random.randint(nks[2], (batch, 1), 0, _N_LOCS)
    neg_a = jax.random.randint(nks[3], (batch, _SEQ), 0, _N_APPS)
    neg_nodes_idx = jnp.concatenate(
        [_N_APPS + _N_LOCS + neg_t, _N_APPS + neg_l, neg_a],
        axis=1).reshape(-1)

    ar = jnp.arange(m, dtype=jnp.int32)
    lp_pos = jnp.full((_NP,), -1, jnp.int32).at[nodes_idx].max(ar)
    lp_neg = jnp.full((_NP,), -1, jnp.int32).at[neg_nodes_idx].max(ar)
    act_p = (lp_pos >= 0).astype(jnp.int32)
    act_n = (lp_neg >= 0).astype(jnp.int32)
    islast_p = jnp.zeros((m,), jnp.int32).at[
        jnp.where(lp_pos >= 0, lp_pos, m)].set(1, mode="drop")
    islast_n = jnp.zeros((m,), jnp.int32).at[
        jnp.where(lp_neg >= 0, lp_neg, m)].set(1, mode="drop")

    src, dst = edge_index[0], edge_index[1]
    e = src.shape[0]
    ep = -(-e // (_NSUB * _CH * 2)) * (_NSUB * _CH * 2)
    pad = ep - e
    src_p = jnp.concatenate([src.astype(jnp.int32),
                             jnp.zeros((pad,), jnp.int32)])
    dst_p = jnp.concatenate([dst.astype(jnp.int32),
                             jnp.full((pad,), _TRASH, jnp.int32)])

    x_pad = jnp.zeros((_NP, _DIM), jnp.float32).at[:_N_NODES].set(tla_emb)
    x_mp = x_pad * act_p.astype(jnp.float32)[:, None]
    x_mn = x_pad * act_n.astype(jnp.float32)[:, None]

    acc_a, deg = _sc_aggregate(x_mp, x_mn, src_p, dst_p,
                               act_p, act_n, True)
    sum1_p, sum1_n = acc_a[0], acc_a[1]
    degf = deg.astype(jnp.float32)
    dg_p = jnp.broadcast_to(degf[0][:, None], (_NP, 16))
    dg_n = jnp.broadcast_to(degf[1][:, None], (_NP, 16))
    ap16 = jnp.broadcast_to(act_p.astype(jnp.float32)[:, None], (_NP, 16))
    an16 = jnp.broadcast_to(act_n.astype(jnp.float32)[:, None], (_NP, 16))

    h1a_p, h1a_n, h1b = _tc_sage(x_pad, sum1_p, sum1_n, dg_p, dg_n,
                                 ap16, an16, Ws1, Wn1, b1)

    acc_b = _sc_aggregate(h1a_p, h1a_n, src_p, dst_p, act_p, act_n, False)
    sum2_p, sum2_n = acc_b[0], acc_b[1]

    h2a_p, h2a_n, h2b = _tc_sage2(h1a_p, h1a_n, h1b, sum2_p, sum2_n,
                                  dg_p, dg_n, Ws2, Wn2, b2)

    tf = jnp.concatenate([h2a_p, h2a_n, h2b], axis=0)
    idx_pos = jnp.where(islast_p == 1, nodes_idx, nodes_idx + 2 * _NP)
    idx_neg = jnp.where(islast_n == 1, neg_nodes_idx + _NP,
                        neg_nodes_idx + 2 * _NP)
    idx_all = jnp.concatenate([idx_pos, idx_neg]).astype(jnp.int32)
    dummy = (jnp.arange(2 * batch, dtype=jnp.int32) % _N_NODES)[:, None]
    idx24 = jnp.concatenate(
        [idx_all.reshape(2 * batch, _SEQ + 2),
         jnp.broadcast_to(dummy, (2 * batch, _GRP - _SEQ - 2))],
        axis=1).reshape(-1)
    g = _sc_gather(tf, idx24)
    loss = _tc_sim(g.reshape(2 * batch, _GRP, _DIM), 2 * batch)
    return loss[:batch], loss[batch:]


# speedup vs baseline: 1.4688x; 1.0044x over previous
import functools

import jax
import jax.numpy as jnp
from jax import lax
from jax.experimental import pallas as pl
from jax.experimental.pallas import tpu as pltpu
from jax.experimental.pallas import tpu_sc as plsc

_N_TIMES = 1000
_N_LOCS = 1000
_N_APPS = 8000
_N_NODES = 10000
_DIM = 128
_SEQ = 20

_NP = 10240
_TRASH = 10100
_CH = 128
_NSUB = 16
_NW = 32
_ROWS_PW = _NP // _NSUB


def _sc_aggregate(table_p, table_n, src, dst, act_p, act_n, with_deg):
    ep = src.shape[0]
    epw = ep // _NSUB
    nch = epw // _CH
    assert nch % 2 == 0
    mesh = plsc.VectorSubcoreMesh(core_axis_name="c", subcore_axis_name="s")

    out_type = [jax.ShapeDtypeStruct((_NW, _ROWS_PW, _DIM), jnp.float32)]
    scratch = [
        pltpu.VMEM((_CH,), jnp.int32),
        pltpu.VMEM((_CH,), jnp.int32),
        pltpu.VMEM((_CH,), jnp.int32),
        pltpu.VMEM((_CH,), jnp.int32),
        pltpu.VMEM((_CH, _DIM), jnp.float32),
        pltpu.VMEM((_CH, _DIM), jnp.float32),
        pltpu.VMEM_SHARED((_NP, _DIM), jnp.float32),
        pltpu.SemaphoreType.DMA,
        pltpu.SemaphoreType.DMA,
        pltpu.SemaphoreType.DMA,
        pltpu.SemaphoreType.DMA,
        pltpu.SemaphoreType.DMA,
        pltpu.SemaphoreType.DMA,
    ]
    if with_deg:
        out_type.append(jax.ShapeDtypeStruct((_NW, _ROWS_PW), jnp.int32))
        scratch += [pltpu.VMEM((_CH,), jnp.int32),
                    pltpu.VMEM((_CH,), jnp.int32),
                    pltpu.SemaphoreType.DMA,
                    pltpu.SemaphoreType.DMA,
                    pltpu.SemaphoreType.DMA,
                    pltpu.SemaphoreType.DMA,
                    pltpu.VMEM_SHARED((_NP,), jnp.int32)]

    @functools.partial(pl.kernel, out_type=out_type, mesh=mesh,
                       scratch_types=scratch)
    def k(tp_h, tn_h, src_h, dst_h, actp_h, actn_h, zeros_h, zeros1_h,
          *rest):
        if with_deg:
            (out_h, deg_out_h, sva, svb, dva, dvb, rva, rvb, acc_sh,
             s_sta, s_stb, s_ra, s_rb, s_wa, s_wb, ava, avb, s_aa, s_ab,
             s_da, s_db, deg_sh) = rest
        else:
            (out_h, sva, svb, dva, dvb, rva, rvb, acc_sh,
             s_sta, s_stb, s_ra, s_rb, s_wa, s_wb) = rest
            ava = avb = s_aa = s_ab = s_da = s_db = deg_sh = None
        c = lax.axis_index("c")
        s = lax.axis_index("s")
        wid = c * _NSUB + s
        pltpu.sync_copy(zeros_h, acc_sh.at[pl.ds(s * _ROWS_PW, _ROWS_PW)])
        if with_deg:
            pltpu.sync_copy(zeros1_h,
                            deg_sh.at[pl.ds(s * _ROWS_PW, _ROWS_PW)])
        plsc.subcore_barrier()
        base = s * epw

        def stage(off, sv, dv, sem):
            pltpu.async_copy(src_h.at[pl.ds(off, _CH)], sv, sem)
            pltpu.async_copy(dst_h.at[pl.ds(off, _CH)], dv, sem)

        def stage_wait(off, sv, dv, sem):
            pltpu.make_async_copy(src_h.at[pl.ds(off, _CH)], sv, sem).wait()
            pltpu.make_async_copy(dst_h.at[pl.ds(off, _CH)], dv, sem).wait()

        def issue_gathers(sv, av, rv, s_a, s_r):
            @pl.when(c == 0)
            def _():
                pltpu.async_copy(tp_h.at[sv], rv, s_r)
                if with_deg:
                    pltpu.async_copy(actp_h.at[sv], av, s_a)

            @pl.when(c == 1)
            def _():
                pltpu.async_copy(tn_h.at[sv], rv, s_r)
                if with_deg:
                    pltpu.async_copy(actn_h.at[sv], av, s_a)

        def finish(sv, dv, av, rv, s_a, s_r, s_w, s_d):
            pltpu.make_async_copy(tp_h.at[sv], rv, s_r).wait()
            pltpu.async_copy(rv, acc_sh.at[dv], s_w, add=True)
            if with_deg:
                pltpu.make_async_copy(actp_h.at[sv], av, s_a).wait()
                pltpu.async_copy(av, deg_sh.at[dv], s_d, add=True)

        def drain(dv, av, rv, s_w, s_d):
            pltpu.make_async_copy(rv, acc_sh.at[dv], s_w).wait()
            if with_deg:
                pltpu.make_async_copy(av, deg_sh.at[dv], s_d).wait()

        def body(t, carry):
            @pl.when(t > 0)
            def _():
                drain(dva, ava, rva, s_wa, s_da)
                drain(dvb, avb, rvb, s_wb, s_db)

            off_a = base + (2 * t) * _CH
            off_b = off_a + _CH
            stage(off_a, sva, dva, s_sta)
            stage(off_b, svb, dvb, s_stb)
            stage_wait(off_a, sva, dva, s_sta)
            issue_gathers(sva, ava, rva, s_aa, s_ra)
            stage_wait(off_b, svb, dvb, s_stb)
            issue_gathers(svb, avb, rvb, s_ab, s_rb)
            finish(sva, dva, ava, rva, s_aa, s_ra, s_wa, s_da)
            finish(svb, dvb, avb, rvb, s_ab, s_rb, s_wb, s_db)
            return carry

        lax.fori_loop(0, nch // 2, body, 0)
        drain(dva, ava, rva, s_wa, s_da)
        drain(dvb, avb, rvb, s_wb, s_db)
        plsc.subcore_barrier()
        pltpu.sync_copy(acc_sh.at[pl.ds(s * _ROWS_PW, _ROWS_PW)],
                        out_h.at[wid])
        if with_deg:
            pltpu.sync_copy(deg_sh.at[pl.ds(s * _ROWS_PW, _ROWS_PW)],
                            deg_out_h.at[wid])

    zeros = jnp.zeros((_ROWS_PW, _DIM), jnp.float32)
    zeros1 = jnp.zeros((_ROWS_PW,), jnp.int32)
    if with_deg:
        out, deg = k(table_p, table_n, src, dst, act_p, act_n, zeros,
                     zeros1)
        return (out.reshape(2, _NP, _DIM), deg.reshape(2, _NP))
    (out,) = k(table_p, table_n, src, dst, act_p, act_n, zeros, zeros1)
    return out.reshape(2, _NP, _DIM)


def _sc_gather(table, idx):
    n = idx.shape[0]
    per_w = n // _NW
    nch = per_w // _CH
    assert nch % 2 == 0
    mesh = plsc.VectorSubcoreMesh(core_axis_name="c", subcore_axis_name="s")

    @functools.partial(
        pl.kernel,
        out_type=jax.ShapeDtypeStruct((n, _DIM), jnp.float32),
        mesh=mesh,
        scratch_types=[
            pltpu.VMEM((_CH,), jnp.int32),
            pltpu.VMEM((_CH,), jnp.int32),
            pltpu.VMEM((_CH, _DIM), jnp.float32),
            pltpu.VMEM((_CH, _DIM), jnp.float32),
            pltpu.SemaphoreType.DMA,
            pltpu.SemaphoreType.DMA,
            pltpu.SemaphoreType.DMA,
            pltpu.SemaphoreType.DMA,
        ],
    )
    def k(table_h, idx_h, out_h, iva, ivb, rva, rvb, s_ia, s_ib, s_ra,
          s_rb):
        c = lax.axis_index("c")
        s = lax.axis_index("s")
        base = (c * _NSUB + s) * per_w

        def body(t, carry):
            off_a = base + (2 * t) * _CH
            off_b = off_a + _CH
            pltpu.async_copy(idx_h.at[pl.ds(off_a, _CH)], iva, s_ia)
            pltpu.async_copy(idx_h.at[pl.ds(off_b, _CH)], ivb, s_ib)
            pltpu.make_async_copy(idx_h.at[pl.ds(off_a, _CH)], iva,
                                  s_ia).wait()
            pltpu.async_copy(table_h.at[iva], rva, s_ra)
            pltpu.make_async_copy(idx_h.at[pl.ds(off_b, _CH)], ivb,
                                  s_ib).wait()
            pltpu.async_copy(table_h.at[ivb], rvb, s_rb)
            pltpu.make_async_copy(table_h.at[iva], rva, s_ra).wait()
            pltpu.sync_copy(rva, out_h.at[pl.ds(off_a, _CH)])
            pltpu.make_async_copy(table_h.at[ivb], rvb, s_rb).wait()
            pltpu.sync_copy(rvb, out_h.at[pl.ds(off_b, _CH)])
            return carry

        lax.fori_loop(0, nch // 2, body, 0)

    return k(table, idx)


_RB = 1024


def _mm_body(x_ref, sp_ref, sn_ref, dp_ref, dn_ref, ap_ref, an_ref,
             ws_ref, wn_ref, b_ref, hap_ref, han_ref, hb_ref):
    x = x_ref[...]
    ws = ws_ref[...]
    wn = wn_ref[...]
    b = b_ref[...]
    s = lax.dot_general(x, ws, (((1,), (1,)), ((), ())),
                        preferred_element_type=jnp.float32)
    np_ = sp_ref[...] / jnp.maximum(dp_ref[...][:, 0:1], 1.0)
    nn_ = sn_ref[...] / jnp.maximum(dn_ref[...][:, 0:1], 1.0)
    mp = lax.dot_general(np_, wn, (((1,), (1,)), ((), ())),
                         preferred_element_type=jnp.float32)
    mn = lax.dot_general(nn_, wn, (((1,), (1,)), ((), ())),
                         preferred_element_type=jnp.float32)
    hap_ref[...] = jnp.maximum(s + mp + b, 0.0) * ap_ref[...][:, 0:1]
    han_ref[...] = jnp.maximum(s + mn + b, 0.0) * an_ref[...][:, 0:1]
    hb_ref[...] = jnp.maximum(s + b, 0.0)


def _tc_sage(x, sum_p, sum_n, deg_p, deg_n, act_p16, act_n16, Ws, Wn, b):
    grid = (_NP // _RB,)
    row = pl.BlockSpec((_RB, _DIM), lambda i: (i, 0))
    dcol = pl.BlockSpec((_RB, 16), lambda i: (i, 0))
    full = pl.BlockSpec((_DIM, _DIM), lambda i: (0, 0))
    bspec = pl.BlockSpec((1, _DIM), lambda i: (0, 0))
    out_sh = jax.ShapeDtypeStruct((_NP, _DIM), jnp.float32)
    return pl.pallas_call(
        _mm_body,
        grid=grid,
        in_specs=[row, row, row, dcol, dcol, dcol, dcol, full, full, bspec],
        out_specs=[row, row, row],
        out_shape=[out_sh, out_sh, out_sh],
    )(x, sum_p, sum_n, deg_p, deg_n, act_p16, act_n16, Ws, Wn,
      b.reshape(1, _DIM))


def _mm2_body(xp_ref, xn_ref, xb_ref, sp_ref, sn_ref, dp_ref, dn_ref,
              ws_ref, wn_ref, b_ref, hap_ref, han_ref, hb_ref):
    ws = ws_ref[...]
    wn = wn_ref[...]
    b = b_ref[...]
    dot = lambda a, w: lax.dot_general(a, w, (((1,), (1,)), ((), ())),
                                       preferred_element_type=jnp.float32)
    np_ = sp_ref[...] / jnp.maximum(dp_ref[...][:, 0:1], 1.0)
    nn_ = sn_ref[...] / jnp.maximum(dn_ref[...][:, 0:1], 1.0)
    hap_ref[...] = jnp.maximum(dot(xp_ref[...], ws) + dot(np_, wn) + b, 0.0)
    han_ref[...] = jnp.maximum(dot(xn_ref[...], ws) + dot(nn_, wn) + b, 0.0)
    hb_ref[...] = jnp.maximum(dot(xb_ref[...], ws) + b, 0.0)


def _tc_sage2(xp, xn, xb, sum_p, sum_n, deg_p, deg_n, Ws, Wn, b):
    grid = (_NP // _RB,)
    row = pl.BlockSpec((_RB, _DIM), lambda i: (i, 0))
    dcol = pl.BlockSpec((_RB, 16), lambda i: (i, 0))
    full = pl.BlockSpec((_DIM, _DIM), lambda i: (0, 0))
    bspec = pl.BlockSpec((1, _DIM), lambda i: (0, 0))
    out_sh = jax.ShapeDtypeStruct((_NP, _DIM), jnp.float32)
    return pl.pallas_call(
        _mm2_body,
        grid=grid,
        in_specs=[row, row, row, row, row, dcol, dcol, full, full, bspec],
        out_specs=[row, row, row],
        out_shape=[out_sh, out_sh, out_sh],
    )(xp, xn, xb, sum_p, sum_n, deg_p, deg_n, Ws, Wn, b.reshape(1, _DIM))


_SB = 128
_GRP = 24


def _sim_body(g_ref, out_ref):
    h = g_ref[...]
    t = h[:, 0, :]
    l = h[:, 1, :]
    a = h[:, 2:_SEQ + 2, :]
    nt2 = jnp.sum(t * t, axis=-1, keepdims=True)
    nl2 = jnp.sum(l * l, axis=-1, keepdims=True)
    stl = jnp.sum(t * l, axis=-1, keepdims=True)
    na2 = jnp.sum(a * a, axis=-1)
    p = jnp.sum(t[:, None, :] * a, axis=-1)
    q = jnp.sum(l[:, None, :] * a, axis=-1)
    nt = jnp.sqrt(nt2)
    nl = jnp.sqrt(nl2)
    ntl = jnp.sqrt(nt2 + 2.0 * stl + nl2)
    nu = jnp.sqrt(nl2 + 2.0 * q + na2)
    nv = jnp.sqrt(nt2 + 2.0 * p + na2)
    na = jnp.sqrt(na2)
    sim_t = jnp.sum((stl + p) / (nt * nu), axis=-1)
    sim_l = jnp.sum((stl + q) / (nl * nv), axis=-1)
    sim_a = jnp.sum((p + q) / (na * ntl), axis=-1)
    out_ref[...] = sim_t + sim_l + sim_a


def _tc_sim(g, nrows):
    grid = (nrows // _SB,)
    return pl.pallas_call(
        _sim_body,
        grid=grid,
        in_specs=[pl.BlockSpec((_SB, _GRP, _DIM), lambda i: (i, 0, 0))],
        out_specs=pl.BlockSpec((_SB,), lambda i: (i,)),
        out_shape=jax.ShapeDtypeStruct((nrows,), jnp.float32),
    )(g)


def kernel(users, times, locs, app_seq, edge_index, tla_emb,
           Ws1, Wn1, b1, Ws2, Wn2, b2):
    batch = users.shape[0]
    m = batch * (_SEQ + 2)
    nodes_idx = jnp.concatenate(
        [_N_APPS + _N_LOCS + times, _N_APPS + locs, app_seq],
        axis=1).reshape(-1)
    nk = jax.random.key(42)
    nks = jax.random.split(nk, 4)
    neg_t = jax.random.randint(nks[1], (batch, 1), 0, _N_TIMES)
    neg_l = jax.random.randint(nks[2], (batch, 1), 0, _N_LOCS)
    neg_a = jax.random.randint(nks[3], (batch, _SEQ), 0, _N_APPS)
    neg_nodes_idx = jnp.concatenate(
        [_N_APPS + _N_LOCS + neg_t, _N_APPS + neg_l, neg_a],
        axis=1).reshape(-1)

    ar = jnp.arange(m, dtype=jnp.int32)
    lp_pos = jnp.full((_NP,), -1, jnp.int32).at[nodes_idx].max(ar)
    lp_neg = jnp.full((_NP,), -1, jnp.int32).at[neg_nodes_idx].max(ar)
    act_p = (lp_pos >= 0).astype(jnp.int32)
    act_n = (lp_neg >= 0).astype(jnp.int32)
    islast_p = jnp.zeros((m,), jnp.int32).at[
        jnp.where(lp_pos >= 0, lp_pos, m)].set(1, mode="drop")
    islast_n = jnp.zeros((m,), jnp.int32).at[
        jnp.where(lp_neg >= 0, lp_neg, m)].set(1, mode="drop")

    src, dst = edge_index[0], edge_index[1]
    e = src.shape[0]
    ep = -(-e // (_NSUB * _CH * 2)) * (_NSUB * _CH * 2)
    pad = ep - e
    src_p = jnp.concatenate([src.astype(jnp.int32),
                             jnp.zeros((pad,), jnp.int32)])
    dst_p = jnp.concatenate([dst.astype(jnp.int32),
                             jnp.full((pad,), _TRASH, jnp.int32)])

    x_pad = jnp.zeros((_NP, _DIM), jnp.float32).at[:_N_NODES].set(tla_emb)
    x_mp = x_pad * act_p.astype(jnp.float32)[:, None]
    x_mn = x_pad * act_n.astype(jnp.float32)[:, None]

    acc_a, deg = _sc_aggregate(x_mp, x_mn, src_p, dst_p,
                               act_p, act_n, True)
    sum1_p, sum1_n = acc_a[0], acc_a[1]
    degf = deg.astype(jnp.float32)
    dg_p = jnp.broadcast_to(degf[0][:, None], (_NP, 16))
    dg_n = jnp.broadcast_to(degf[1][:, None], (_NP, 16))
    ap16 = jnp.broadcast_to(act_p.astype(jnp.float32)[:, None], (_NP, 16))
    an16 = jnp.broadcast_to(act_n.astype(jnp.float32)[:, None], (_NP, 16))

    h1a_p, h1a_n, h1b = _tc_sage(x_pad, sum1_p, sum1_n, dg_p, dg_n,
                                 ap16, an16, Ws1, Wn1, b1)

    acc_b = _sc_aggregate(h1a_p, h1a_n, src_p, dst_p, act_p, act_n, False)
    sum2_p, sum2_n = acc_b[0], acc_b[1]

    h2a_p, h2a_n, h2b = _tc_sage2(h1a_p, h1a_n, h1b, sum2_p, sum2_n,
                                  dg_p, dg_n, Ws2, Wn2, b2)

    tf = jnp.concatenate([h2a_p, h2a_n, h2b], axis=0)
    idx_pos = jnp.where(islast_p == 1, nodes_idx, nodes_idx + 2 * _NP)
    idx_neg = jnp.where(islast_n == 1, neg_nodes_idx + _NP,
                        neg_nodes_idx + 2 * _NP)
    idx_all = jnp.concatenate([idx_pos, idx_neg]).astype(jnp.int32)
    dummy = (jnp.arange(2 * batch, dtype=jnp.int32) % _N_NODES)[:, None]
    idx24 = jnp.concatenate(
        [idx_all.reshape(2 * batch, _SEQ + 2),
         jnp.broadcast_to(dummy, (2 * batch, _GRP - _SEQ - 2))],
        axis=1).reshape(-1)
    g = _sc_gather(tf, idx24)
    loss = _tc_sim(g.reshape(2 * batch, _GRP, _DIM), 2 * batch)
    return loss[:batch], loss[batch:]


# speedup vs baseline: 1.5364x; 1.0460x over previous
import functools

import jax
import jax.numpy as jnp
from jax import lax
from jax.experimental import pallas as pl
from jax.experimental.pallas import tpu as pltpu
from jax.experimental.pallas import tpu_sc as plsc

_N_TIMES = 1000
_N_LOCS = 1000
_N_APPS = 8000
_N_NODES = 10000
_DIM = 128
_SEQ = 20

_NP = 10240
_TRASH = 10100
_CH = 128
_NSUB = 16
_NW = 32
_ROWS_PW = _NP // _NSUB


def _sc_aggregate(table_p, table_n, src, dst, act_p, act_n, with_deg):
    ep = src.shape[0]
    epw = ep // _NSUB
    nch = epw // _CH
    assert nch % 2 == 0
    mesh = plsc.VectorSubcoreMesh(core_axis_name="c", subcore_axis_name="s")

    out_type = [jax.ShapeDtypeStruct((_NW, _ROWS_PW, _DIM), jnp.float32)]
    scratch = [
        pltpu.VMEM((_CH,), jnp.int32),
        pltpu.VMEM((_CH,), jnp.int32),
        pltpu.VMEM((_CH,), jnp.int32),
        pltpu.VMEM((_CH,), jnp.int32),
        pltpu.VMEM((_CH, _DIM), jnp.float32),
        pltpu.VMEM((_CH, _DIM), jnp.float32),
        pltpu.VMEM_SHARED((_NP, _DIM), jnp.float32),
        pltpu.SemaphoreType.DMA,
        pltpu.SemaphoreType.DMA,
        pltpu.SemaphoreType.DMA,
        pltpu.SemaphoreType.DMA,
        pltpu.SemaphoreType.DMA,
        pltpu.SemaphoreType.DMA,
    ]
    if with_deg:
        out_type.append(jax.ShapeDtypeStruct((_NW, _ROWS_PW), jnp.int32))
        scratch += [pltpu.VMEM((_CH,), jnp.int32),
                    pltpu.VMEM((_CH,), jnp.int32),
                    pltpu.SemaphoreType.DMA,
                    pltpu.SemaphoreType.DMA,
                    pltpu.SemaphoreType.DMA,
                    pltpu.SemaphoreType.DMA,
                    pltpu.VMEM_SHARED((_NP,), jnp.int32)]

    @functools.partial(pl.kernel, out_type=out_type, mesh=mesh,
                       scratch_types=scratch)
    def k(tp_h, tn_h, src_h, dst_h, actp_h, actn_h, zeros_h, zeros1_h,
          *rest):
        if with_deg:
            (out_h, deg_out_h, sva, svb, dva, dvb, rva, rvb, acc_sh,
             s_sta, s_stb, s_ra, s_rb, s_wa, s_wb, ava, avb, s_aa, s_ab,
             s_da, s_db, deg_sh) = rest
        else:
            (out_h, sva, svb, dva, dvb, rva, rvb, acc_sh,
             s_sta, s_stb, s_ra, s_rb, s_wa, s_wb) = rest
            ava = avb = s_aa = s_ab = s_da = s_db = deg_sh = None
        c = lax.axis_index("c")
        s = lax.axis_index("s")
        wid = c * _NSUB + s
        pltpu.sync_copy(zeros_h, acc_sh.at[pl.ds(s * _ROWS_PW, _ROWS_PW)])
        if with_deg:
            pltpu.sync_copy(zeros1_h,
                            deg_sh.at[pl.ds(s * _ROWS_PW, _ROWS_PW)])
        plsc.subcore_barrier()
        base = s * epw

        def stage(off, sv, dv, sem):
            pltpu.async_copy(src_h.at[pl.ds(off, _CH)], sv, sem)
            pltpu.async_copy(dst_h.at[pl.ds(off, _CH)], dv, sem)

        def stage_wait(off, sv, dv, sem):
            pltpu.make_async_copy(src_h.at[pl.ds(off, _CH)], sv, sem).wait()
            pltpu.make_async_copy(dst_h.at[pl.ds(off, _CH)], dv, sem).wait()

        def issue_gathers(sv, av, rv, s_a, s_r):
            @pl.when(c == 0)
            def _():
                pltpu.async_copy(tp_h.at[sv], rv, s_r)
                if with_deg:
                    pltpu.async_copy(actp_h.at[sv], av, s_a)

            @pl.when(c == 1)
            def _():
                pltpu.async_copy(tn_h.at[sv], rv, s_r)
                if with_deg:
                    pltpu.async_copy(actn_h.at[sv], av, s_a)

        def finish(sv, dv, av, rv, s_a, s_r, s_w, s_d):
            pltpu.make_async_copy(tp_h.at[sv], rv, s_r).wait()
            pltpu.async_copy(rv, acc_sh.at[dv], s_w, add=True)
            if with_deg:
                pltpu.make_async_copy(actp_h.at[sv], av, s_a).wait()
                pltpu.async_copy(av, deg_sh.at[dv], s_d, add=True)

        def drain(dv, av, rv, s_w, s_d):
            pltpu.make_async_copy(rv, acc_sh.at[dv], s_w).wait()
            if with_deg:
                pltpu.make_async_copy(av, deg_sh.at[dv], s_d).wait()

        def body(t, carry):
            @pl.when(t > 0)
            def _():
                drain(dva, ava, rva, s_wa, s_da)
                drain(dvb, avb, rvb, s_wb, s_db)

            off_a = base + (2 * t) * _CH
            off_b = off_a + _CH
            stage(off_a, sva, dva, s_sta)
            stage(off_b, svb, dvb, s_stb)
            stage_wait(off_a, sva, dva, s_sta)
            issue_gathers(sva, ava, rva, s_aa, s_ra)
            stage_wait(off_b, svb, dvb, s_stb)
            issue_gathers(svb, avb, rvb, s_ab, s_rb)
            finish(sva, dva, ava, rva, s_aa, s_ra, s_wa, s_da)
            finish(svb, dvb, avb, rvb, s_ab, s_rb, s_wb, s_db)
            return carry

        lax.fori_loop(0, nch // 2, body, 0)
        drain(dva, ava, rva, s_wa, s_da)
        drain(dvb, avb, rvb, s_wb, s_db)
        plsc.subcore_barrier()
        pltpu.sync_copy(acc_sh.at[pl.ds(s * _ROWS_PW, _ROWS_PW)],
                        out_h.at[wid])
        if with_deg:
            pltpu.sync_copy(deg_sh.at[pl.ds(s * _ROWS_PW, _ROWS_PW)],
                            deg_out_h.at[wid])

    zeros = jnp.zeros((_ROWS_PW, _DIM), jnp.float32)
    zeros1 = jnp.zeros((_ROWS_PW,), jnp.int32)
    if with_deg:
        out, deg = k(table_p, table_n, src, dst, act_p, act_n, zeros,
                     zeros1)
        return (out.reshape(2, _NP, _DIM), deg.reshape(2, _NP))
    (out,) = k(table_p, table_n, src, dst, act_p, act_n, zeros, zeros1)
    return out.reshape(2, _NP, _DIM)


def _sc_gather(table, idx):
    n = idx.shape[0]
    per_w = n // _NW
    nch = per_w // _CH
    assert nch % 2 == 0
    mesh = plsc.VectorSubcoreMesh(core_axis_name="c", subcore_axis_name="s")

    @functools.partial(
        pl.kernel,
        out_type=jax.ShapeDtypeStruct((n, _DIM), jnp.float32),
        mesh=mesh,
        scratch_types=[
            pltpu.VMEM((_CH,), jnp.int32),
            pltpu.VMEM((_CH,), jnp.int32),
            pltpu.VMEM((_CH, _DIM), jnp.float32),
            pltpu.VMEM((_CH, _DIM), jnp.float32),
            pltpu.SemaphoreType.DMA,
            pltpu.SemaphoreType.DMA,
            pltpu.SemaphoreType.DMA,
            pltpu.SemaphoreType.DMA,
        ],
    )
    def k(table_h, idx_h, out_h, iva, ivb, rva, rvb, s_ia, s_ib, s_ra,
          s_rb):
        c = lax.axis_index("c")
        s = lax.axis_index("s")
        base = (c * _NSUB + s) * per_w

        def body(t, carry):
            off_a = base + (2 * t) * _CH
            off_b = off_a + _CH
            pltpu.async_copy(idx_h.at[pl.ds(off_a, _CH)], iva, s_ia)
            pltpu.async_copy(idx_h.at[pl.ds(off_b, _CH)], ivb, s_ib)
            pltpu.make_async_copy(idx_h.at[pl.ds(off_a, _CH)], iva,
                                  s_ia).wait()
            pltpu.async_copy(table_h.at[iva], rva, s_ra)
            pltpu.make_async_copy(idx_h.at[pl.ds(off_b, _CH)], ivb,
                                  s_ib).wait()
            pltpu.async_copy(table_h.at[ivb], rvb, s_rb)
            pltpu.make_async_copy(table_h.at[iva], rva, s_ra).wait()
            pltpu.sync_copy(rva, out_h.at[pl.ds(off_a, _CH)])
            pltpu.make_async_copy(table_h.at[ivb], rvb, s_rb).wait()
            pltpu.sync_copy(rvb, out_h.at[pl.ds(off_b, _CH)])
            return carry

        lax.fori_loop(0, nch // 2, body, 0)

    return k(table, idx)


_RB = 1024


def _mm_body(x_ref, sp_ref, sn_ref, dp_ref, dn_ref, ap_ref, an_ref,
             ws_ref, wn_ref, b_ref, hap_ref, han_ref, hb_ref):
    x = x_ref[...]
    ws = ws_ref[...]
    wn = wn_ref[...]
    b = b_ref[...]
    s = lax.dot_general(x, ws, (((1,), (1,)), ((), ())),
                        preferred_element_type=jnp.float32)
    np_ = sp_ref[...] / jnp.maximum(dp_ref[...][:, 0:1], 1.0)
    nn_ = sn_ref[...] / jnp.maximum(dn_ref[...][:, 0:1], 1.0)
    mp = lax.dot_general(np_, wn, (((1,), (1,)), ((), ())),
                         preferred_element_type=jnp.float32)
    mn = lax.dot_general(nn_, wn, (((1,), (1,)), ((), ())),
                         preferred_element_type=jnp.float32)
    hap_ref[...] = jnp.maximum(s + mp + b, 0.0) * ap_ref[...][:, 0:1]
    han_ref[...] = jnp.maximum(s + mn + b, 0.0) * an_ref[...][:, 0:1]
    hb_ref[...] = jnp.maximum(s + b, 0.0)


def _tc_sage(x, sum_p, sum_n, deg_p, deg_n, act_p16, act_n16, Ws, Wn, b):
    grid = (_NP // _RB,)
    row = pl.BlockSpec((_RB, _DIM), lambda i: (i, 0))
    dcol = pl.BlockSpec((_RB, 16), lambda i: (i, 0))
    full = pl.BlockSpec((_DIM, _DIM), lambda i: (0, 0))
    bspec = pl.BlockSpec((1, _DIM), lambda i: (0, 0))
    out_sh = jax.ShapeDtypeStruct((_NP, _DIM), jnp.float32)
    return pl.pallas_call(
        _mm_body,
        grid=grid,
        in_specs=[row, row, row, dcol, dcol, dcol, dcol, full, full, bspec],
        out_specs=[row, row, row],
        out_shape=[out_sh, out_sh, out_sh],
    )(x, sum_p, sum_n, deg_p, deg_n, act_p16, act_n16, Ws, Wn,
      b.reshape(1, _DIM))


def _mm2_body(xp_ref, xn_ref, xb_ref, sp_ref, sn_ref, dp_ref, dn_ref,
              ws_ref, wn_ref, b_ref, hap_ref, han_ref, hb_ref):
    ws = ws_ref[...]
    wn = wn_ref[...]
    b = b_ref[...]
    dot = lambda a, w: lax.dot_general(a, w, (((1,), (1,)), ((), ())),
                                       preferred_element_type=jnp.float32)
    np_ = sp_ref[...] / jnp.maximum(dp_ref[...][:, 0:1], 1.0)
    nn_ = sn_ref[...] / jnp.maximum(dn_ref[...][:, 0:1], 1.0)
    hap_ref[...] = jnp.maximum(dot(xp_ref[...], ws) + dot(np_, wn) + b, 0.0)
    han_ref[...] = jnp.maximum(dot(xn_ref[...], ws) + dot(nn_, wn) + b, 0.0)
    hb_ref[...] = jnp.maximum(dot(xb_ref[...], ws) + b, 0.0)


def _tc_sage2(xp, xn, xb, sum_p, sum_n, deg_p, deg_n, Ws, Wn, b):
    grid = (_NP // _RB,)
    row = pl.BlockSpec((_RB, _DIM), lambda i: (i, 0))
    dcol = pl.BlockSpec((_RB, 16), lambda i: (i, 0))
    full = pl.BlockSpec((_DIM, _DIM), lambda i: (0, 0))
    bspec = pl.BlockSpec((1, _DIM), lambda i: (0, 0))
    out_sh = jax.ShapeDtypeStruct((_NP, _DIM), jnp.float32)
    return pl.pallas_call(
        _mm2_body,
        grid=grid,
        in_specs=[row, row, row, row, row, dcol, dcol, full, full, bspec],
        out_specs=[row, row, row],
        out_shape=[out_sh, out_sh, out_sh],
    )(xp, xn, xb, sum_p, sum_n, deg_p, deg_n, Ws, Wn, b.reshape(1, _DIM))


_SB = 128
_GRP = 24


def _sim_body(g_ref, out_ref):
    h = g_ref[...]
    t = h[:, 0, :]
    l = h[:, 1, :]
    a = h[:, 2:_SEQ + 2, :]
    nt2 = jnp.sum(t * t, axis=-1, keepdims=True)
    nl2 = jnp.sum(l * l, axis=-1, keepdims=True)
    stl = jnp.sum(t * l, axis=-1, keepdims=True)
    na2 = jnp.sum(a * a, axis=-1)
    p = jnp.sum(t[:, None, :] * a, axis=-1)
    q = jnp.sum(l[:, None, :] * a, axis=-1)
    nt = jnp.sqrt(nt2)
    nl = jnp.sqrt(nl2)
    ntl = jnp.sqrt(nt2 + 2.0 * stl + nl2)
    nu = jnp.sqrt(nl2 + 2.0 * q + na2)
    nv = jnp.sqrt(nt2 + 2.0 * p + na2)
    na = jnp.sqrt(na2)
    sim_t = jnp.sum((stl + p) / (nt * nu), axis=-1)
    sim_l = jnp.sum((stl + q) / (nl * nv), axis=-1)
    sim_a = jnp.sum((p + q) / (na * ntl), axis=-1)
    out_ref[...] = sim_t + sim_l + sim_a


def _tc_sim(g, nrows):
    grid = (nrows // _SB,)
    return pl.pallas_call(
        _sim_body,
        grid=grid,
        in_specs=[pl.BlockSpec((_SB, _GRP, _DIM), lambda i: (i, 0, 0))],
        out_specs=pl.BlockSpec((_SB,), lambda i: (i,)),
        out_shape=jax.ShapeDtypeStruct((nrows,), jnp.float32),
    )(g)


def kernel(users, times, locs, app_seq, edge_index, tla_emb,
           Ws1, Wn1, b1, Ws2, Wn2, b2):
    batch = users.shape[0]
    m = batch * (_SEQ + 2)
    nodes_idx = jnp.concatenate(
        [_N_APPS + _N_LOCS + times, _N_APPS + locs, app_seq],
        axis=1).reshape(-1)
    with jax.ensure_compile_time_eval():
        nk = jax.random.key(42)
        nks = jax.random.split(nk, 4)
        neg_t = jax.random.randint(nks[1], (batch, 1), 0, _N_TIMES)
        neg_l = jax.random.randint(nks[2], (batch, 1), 0, _N_LOCS)
        neg_a = jax.random.randint(nks[3], (batch, _SEQ), 0, _N_APPS)
        neg_nodes_idx = jnp.concatenate(
            [_N_APPS + _N_LOCS + neg_t, _N_APPS + neg_l, neg_a],
            axis=1).reshape(-1)
        ar = jnp.arange(m, dtype=jnp.int32)
        lp_neg = jnp.full((_NP,), -1, jnp.int32).at[neg_nodes_idx].max(ar)
        act_n = (lp_neg >= 0).astype(jnp.int32)
        islast_n = jnp.zeros((m,), jnp.int32).at[
            jnp.where(lp_neg >= 0, lp_neg, m)].set(1, mode="drop")
        idx_neg_const = jnp.where(islast_n == 1, neg_nodes_idx + _NP,
                                  neg_nodes_idx + 2 * _NP).astype(jnp.int32)

    lp_pos = jnp.full((_NP,), -1, jnp.int32).at[nodes_idx].max(ar)
    act_p = (lp_pos >= 0).astype(jnp.int32)
    islast_p = jnp.zeros((m,), jnp.int32).at[
        jnp.where(lp_pos >= 0, lp_pos, m)].set(1, mode="drop")

    src, dst = edge_index[0], edge_index[1]
    e = src.shape[0]
    ep = -(-e // (_NSUB * _CH * 2)) * (_NSUB * _CH * 2)
    pad = ep - e
    src_p = jnp.concatenate([src.astype(jnp.int32),
                             jnp.zeros((pad,), jnp.int32)])
    dst_p = jnp.concatenate([dst.astype(jnp.int32),
                             jnp.full((pad,), _TRASH, jnp.int32)])

    x_pad = jnp.zeros((_NP, _DIM), jnp.float32).at[:_N_NODES].set(tla_emb)
    x_mp = x_pad * act_p.astype(jnp.float32)[:, None]
    x_mn = x_pad * act_n.astype(jnp.float32)[:, None]

    acc_a, deg = _sc_aggregate(x_mp, x_mn, src_p, dst_p,
                               act_p, act_n, True)
    sum1_p, sum1_n = acc_a[0], acc_a[1]
    degf = deg.astype(jnp.float32)
    dg_p = jnp.broadcast_to(degf[0][:, None], (_NP, 16))
    dg_n = jnp.broadcast_to(degf[1][:, None], (_NP, 16))
    ap16 = jnp.broadcast_to(act_p.astype(jnp.float32)[:, None], (_NP, 16))
    an16 = jnp.broadcast_to(act_n.astype(jnp.float32)[:, None], (_NP, 16))

    h1a_p, h1a_n, h1b = _tc_sage(x_pad, sum1_p, sum1_n, dg_p, dg_n,
                                 ap16, an16, Ws1, Wn1, b1)

    acc_b = _sc_aggregate(h1a_p, h1a_n, src_p, dst_p, act_p, act_n, False)
    sum2_p, sum2_n = acc_b[0], acc_b[1]

    h2a_p, h2a_n, h2b = _tc_sage2(h1a_p, h1a_n, h1b, sum2_p, sum2_n,
                                  dg_p, dg_n, Ws2, Wn2, b2)

    tf = jnp.concatenate([h2a_p, h2a_n, h2b], axis=0)
    idx_pos = jnp.where(islast_p == 1, nodes_idx, nodes_idx + 2 * _NP)
    idx_all = jnp.concatenate([idx_pos.astype(jnp.int32), idx_neg_const])
    dummy = (jnp.arange(2 * batch, dtype=jnp.int32) % _N_NODES)[:, None]
    idx24 = jnp.concatenate(
        [idx_all.reshape(2 * batch, _SEQ + 2),
         jnp.broadcast_to(dummy, (2 * batch, _GRP - _SEQ - 2))],
        axis=1).reshape(-1)
    g = _sc_gather(tf, idx24)
    loss = _tc_sim(g.reshape(2 * batch, _GRP, _DIM), 2 * batch)
    return loss[:batch], loss[batch:]
